# baseline (speedup 1.0000x reference)
import jax
import jax.numpy as jnp
from jax.experimental import pallas as pl
from jax.experimental.pallas import tpu as pltpu
from jax.experimental.pallas import tpu_sc as plsc

N = 4096
C = 64
D = 128
M = 16
H = 4
HD = 32
DFF = 256
B = 256
NB = N // B
PAD = N
TBL = N + 8
BIG = 1 << 30
SCALE = 1.0 / (HD ** 0.5)


def _prep_body(coords_ref, coordsT_ref, feat_ref, pw1, pb1, pw2, pb2, prw, prb,
               src_ref, idx_ref):
    cb = coords_ref[...]
    crt = cb.astype(jnp.float32) * (1.0 / 39.0)
    h1 = jnp.maximum(
        jnp.dot(crt, pw1[...], preferred_element_type=jnp.float32) + pb1[...], 0.0)
    pe = jnp.dot(h1, pw2[...], preferred_element_type=jnp.float32) + pb2[...]
    src_ref[...] = (
        jnp.dot(feat_ref[...], prw[...], preferred_element_type=jnp.float32)
        + prb[...] + pe)

    cT = coordsT_ref[...]
    d = jnp.abs(cb[:, 0:1] - cT[0:1, :])
    d = d + jnp.abs(cb[:, 1:2] - cT[1:2, :])
    d = d + jnp.abs(cb[:, 2:3] - cT[2:3, :])
    j = jax.lax.broadcasted_iota(jnp.int32, (B, N), 1)
    key = jnp.where(d <= 4, d * N + j, BIG)
    for m in range(M):
        cur = jnp.min(key, axis=1, keepdims=True)
        idx_ref[:, m:m + 1] = jnp.where(cur < BIG,
                                        jnp.bitwise_and(cur, N - 1),
                                        PAD)
        key = jnp.where(key == cur, BIG, key)


NW = 32
CHUNK = 64
NBUF = 8
GTOT = N * M + N
ROWS_W = GTOT // NW
NCH = ROWS_W // CHUNK


def _sc_gather(table, gidx):
    mesh = plsc.VectorSubcoreMesh(core_axis_name="c", subcore_axis_name="s")

    @pl.kernel(out_type=jax.ShapeDtypeStruct((GTOT, D), jnp.float32),
               mesh=mesh,
               scratch_types=[pltpu.VMEM((ROWS_W,), jnp.int32),
                              pltpu.VMEM((NBUF, CHUNK, D), jnp.float32),
                              pltpu.SemaphoreType.DMA((NBUF,)),
                              pltpu.SemaphoreType.DMA((NBUF,))])
    def kern(x_hbm, i_hbm, o_hbm, idx_v, rows, gsem, wsem):
        wid = jax.lax.axis_index("s") * 2 + jax.lax.axis_index("c")
        base = wid * ROWS_W
        pltpu.sync_copy(i_hbm.at[pl.ds(base, ROWS_W)], idx_v)
        wb = [None] * NBUF
        for g in range(0, NCH, NBUF):
            fired = []
            for s in range(g, min(g + NBUF, NCH)):
                b = s % NBUF
                if wb[b] is not None:
                    wb[b].wait()
                h = pltpu.async_copy(
                    x_hbm.at[idx_v.at[pl.ds(s * CHUNK, CHUNK)]],
                    rows.at[b], gsem.at[b])
                fired.append((s, b, h))
            for s, b, h in fired:
                h.wait()
                wb[b] = pltpu.async_copy(
                    rows.at[b], o_hbm.at[pl.ds(base + s * CHUNK, CHUNK)],
                    wsem.at[b])
        for b in range(NBUF):
            if wb[b] is not None:
                wb[b].wait()

    return kern(table, gidx)


def _main_body(neigh_ref, q_ref, feat_ref, Wq, bq, Wk, bk, Wv, bv, Wo, bo,
               w1, b1, w2, b2, lg, lb, fw1, fb1, fw2, fb2, out_ref):
    n2 = neigh_ref[...].reshape(B * M, D)
    K2 = jnp.dot(n2, Wk[...], preferred_element_type=jnp.float32) + bk[...]
    V2 = jnp.dot(n2, Wv[...], preferred_element_type=jnp.float32) + bv[...]
    q = jnp.dot(q_ref[...], Wq[...], preferred_element_type=jnp.float32) + bq[...]

    P = (K2.reshape(B, M, D) * q.reshape(B, 1, D)).reshape(B * M, D)
    di = jax.lax.broadcasted_iota(jnp.int32, (D, H), 0) // HD
    hi = jax.lax.broadcasted_iota(jnp.int32, (D, H), 1)
    hsel = jnp.where(di == hi, SCALE, 0.0).astype(jnp.float32)
    s3 = jnp.dot(P, hsel, preferred_element_type=jnp.float32).reshape(B, M, H)
    mx = jnp.max(s3, axis=1, keepdims=True)
    e = jnp.exp(s3 - mx)
    attn = e / jnp.sum(e, axis=1, keepdims=True)

    hi2 = jax.lax.broadcasted_iota(jnp.int32, (H, D), 0)
    di2 = jax.lax.broadcasted_iota(jnp.int32, (H, D), 1) // HD
    expand = jnp.where(hi2 == di2, 1.0, 0.0).astype(jnp.float32)
    A128 = jnp.dot(attn.reshape(B * M, H), expand,
                   preferred_element_type=jnp.float32)
    ctx = jnp.sum((A128 * V2).reshape(B, M, D), axis=1)

    tgt = jnp.dot(ctx, Wo[...], preferred_element_type=jnp.float32) + bo[...]
    a1 = jnp.maximum(
        jnp.dot(tgt, w1[...], preferred_element_type=jnp.float32) + b1[...], 0.0)
    hh = tgt + jnp.dot(a1, w2[...], preferred_element_type=jnp.float32) + b2[...]
    mu = jnp.mean(hh, axis=1, keepdims=True)
    dv = hh - mu
    var = jnp.mean(dv * dv, axis=1, keepdims=True)
    y = dv * jax.lax.rsqrt(var + 1e-5) * lg[...] + lb[...]

    t = jnp.dot(y, fw1[...], preferred_element_type=jnp.float32) + fb1[...]
    out_ref[...] = (
        jnp.dot(feat_ref[...], fw2[0:C, :], preferred_element_type=jnp.float32)
        + jnp.dot(t, fw2[C:2 * C, :], preferred_element_type=jnp.float32)
        + fb2[...])


def _bnorm_body(f_ref, g_ref, b_ref, out_ref):
    f = f_ref[...]
    mu = jnp.mean(f, axis=0, keepdims=True)
    dv = f - mu
    var = jnp.mean(dv * dv, axis=0, keepdims=True)
    out_ref[...] = jnp.maximum(
        dv * jax.lax.rsqrt(var + 1e-5) * g_ref[...] + b_ref[...], 0.0)


def _full(shape):
    return pl.BlockSpec(shape, lambda *_: tuple(0 for _ in shape))


def kernel(features, voxel_coords, pe_w1, pe_b1, pe_w2, pe_b2, proj_w, proj_b,
           Wq, bq, Wk, bk, Wv, bv, Wo, bo,
           ffn_w1, ffn_b1, ffn_w2, ffn_b2, ln_g, ln_b,
           fus_w1, fus_b1, fus_w2, fus_b2, bn_g, bn_b):
    r1 = lambda v: v.reshape(1, -1)
    coordsT = voxel_coords.T

    src, idx = pl.pallas_call(
        _prep_body,
        grid=(NB,),
        in_specs=[
            pl.BlockSpec((B, 3), lambda i: (i, 0)),
            _full((3, N)),
            pl.BlockSpec((B, C), lambda i: (i, 0)),
            _full(pe_w1.shape), _full((1, D // 2)),
            _full(pe_w2.shape), _full((1, D)),
            _full(proj_w.shape), _full((1, D)),
        ],
        out_specs=[
            pl.BlockSpec((B, D), lambda i: (i, 0)),
            pl.BlockSpec((B, M), lambda i: (i, 0)),
        ],
        out_shape=[
            jax.ShapeDtypeStruct((N, D), jnp.float32),
            jax.ShapeDtypeStruct((N, M), jnp.int32),
        ],
    )(voxel_coords, coordsT, features, pe_w1, r1(pe_b1), pe_w2, r1(pe_b2),
      proj_w, r1(proj_b))

    table = jnp.concatenate([src, jnp.zeros((TBL - N, D), jnp.float32)], axis=0)
    kvidx = idx.reshape(M, N).T
    gidx = jnp.concatenate([kvidx.reshape(N * M), idx[:, 0]])
    gout = _sc_gather(table, gidx)
    neigh, qrows = gout[:N * M], gout[N * M:]

    fused = pl.pallas_call(
        _main_body,
        grid=(NB,),
        in_specs=[
            pl.BlockSpec((B, M, D), lambda i: (i, 0, 0)),
            pl.BlockSpec((B, D), lambda i: (i, 0)),
            pl.BlockSpec((B, C), lambda i: (i, 0)),
            _full(Wq.shape), _full((1, D)),
            _full(Wk.shape), _full((1, D)),
            _full(Wv.shape), _full((1, D)),
            _full(Wo.shape), _full((1, D)),
            _full(ffn_w1.shape), _full((1, DFF)),
            _full(ffn_w2.shape), _full((1, D)),
            _full((1, D)), _full((1, D)),
            _full(fus_w1.shape), _full((1, C)),
            _full(fus_w2.shape), _full((1, C)),
        ],
        out_specs=pl.BlockSpec((B, C), lambda i: (i, 0)),
        out_shape=jax.ShapeDtypeStruct((N, C), jnp.float32),
    )(neigh.reshape(N, M, D), qrows, features,
      Wq, r1(bq), Wk, r1(bk), Wv, r1(bv), Wo, r1(bo),
      ffn_w1, r1(ffn_b1), ffn_w2, r1(ffn_b2), r1(ln_g), r1(ln_b),
      fus_w1, r1(fus_b1), fus_w2, r1(fus_b2))

    out = pl.pallas_call(
        _bnorm_body,
        in_specs=[_full((N, C)), _full((1, C)), _full((1, C))],
        out_specs=_full((N, C)),
        out_shape=jax.ShapeDtypeStruct((N, C), jnp.float32),
    )(fused, r1(bn_g), r1(bn_b))
    return out


# speedup vs baseline: 4.6262x; 4.6262x over previous
import jax
import jax.numpy as jnp
from jax.experimental import pallas as pl
from jax.experimental.pallas import tpu as pltpu
from jax.experimental.pallas import tpu_sc as plsc

N = 4096
C = 64
D = 128
M = 16
H = 4
HD = 32
DFF = 256
B = 256
NB = N // B
PAD = N
TBL = N + 8
BIG = 1 << 30
SCALE = 1.0 / (HD ** 0.5)


def _prep_body(coords_ref, coordsT_ref, feat_ref, pw1, pb1, pw2, pb2, prw, prb,
               src_ref, idx_ref):
    cb = coords_ref[...]
    crt = cb.astype(jnp.float32) * (1.0 / 39.0)
    h1 = jnp.maximum(
        jnp.dot(crt, pw1[...], preferred_element_type=jnp.float32) + pb1[...], 0.0)
    pe = jnp.dot(h1, pw2[...], preferred_element_type=jnp.float32) + pb2[...]
    src_ref[...] = (
        jnp.dot(feat_ref[...], prw[...], preferred_element_type=jnp.float32)
        + prb[...] + pe)

    cT = coordsT_ref[...]
    d = jnp.abs(cb[:, 0:1] - cT[0:1, :])
    d = d + jnp.abs(cb[:, 1:2] - cT[1:2, :])
    d = d + jnp.abs(cb[:, 2:3] - cT[2:3, :])
    j = jax.lax.broadcasted_iota(jnp.int32, (B, N), 1)
    key = jnp.where(d <= 4, d * N + j, BIG)
    for m in range(M):
        cur = jnp.min(key, axis=1, keepdims=True)
        idx_ref[:, m:m + 1] = jnp.where(cur < BIG,
                                        jnp.bitwise_and(cur, N - 1),
                                        PAD)
        key = jnp.where(key == cur, BIG, key)


NW = 32
CHUNK = 64
NBUF = 8
GTOT = N * M + N
ROWS_W = GTOT // NW
NCH = ROWS_W // CHUNK


def _sc_gather(table, gidx):
    mesh = plsc.VectorSubcoreMesh(core_axis_name="c", subcore_axis_name="s")

    @pl.kernel(out_type=jax.ShapeDtypeStruct((GTOT, D), jnp.float32),
               mesh=mesh,
               scratch_types=[pltpu.VMEM((ROWS_W,), jnp.int32),
                              pltpu.VMEM((NBUF, CHUNK, D), jnp.float32),
                              pltpu.VMEM_SHARED((TBL, D), jnp.float32),
                              pltpu.SemaphoreType.DMA((NBUF,)),
                              pltpu.SemaphoreType.DMA((NBUF,))])
    def kern(x_hbm, i_hbm, o_hbm, idx_v, rows, tbl_spm, gsem, wsem):
        sid = jax.lax.axis_index("s")
        wid = sid * 2 + jax.lax.axis_index("c")
        base = wid * ROWS_W

        @pl.when(sid == 0)
        def _():
            pltpu.sync_copy(x_hbm, tbl_spm)

        plsc.subcore_barrier()
        pltpu.sync_copy(i_hbm.at[pl.ds(base, ROWS_W)], idx_v)
        wb = [None] * NBUF
        for g in range(0, NCH, NBUF):
            fired = []
            for s in range(g, min(g + NBUF, NCH)):
                b = s % NBUF
                if wb[b] is not None:
                    wb[b].wait()
                h = pltpu.async_copy(
                    tbl_spm.at[idx_v.at[pl.ds(s * CHUNK, CHUNK)]],
                    rows.at[b], gsem.at[b])
                fired.append((s, b, h))
            for s, b, h in fired:
                h.wait()
                wb[b] = pltpu.async_copy(
                    rows.at[b], o_hbm.at[pl.ds(base + s * CHUNK, CHUNK)],
                    wsem.at[b])
        for b in range(NBUF):
            if wb[b] is not None:
                wb[b].wait()

    return kern(table, gidx)


def _main_body(neigh_ref, q_ref, feat_ref, Wq, bq, Wk, bk, Wv, bv, Wo, bo,
               w1, b1, w2, b2, lg, lb, fw1, fb1, fw2, fb2, out_ref):
    n2 = neigh_ref[...].reshape(B * M, D)
    K2 = jnp.dot(n2, Wk[...], preferred_element_type=jnp.float32) + bk[...]
    V2 = jnp.dot(n2, Wv[...], preferred_element_type=jnp.float32) + bv[...]
    q = jnp.dot(q_ref[...], Wq[...], preferred_element_type=jnp.float32) + bq[...]

    P = (K2.reshape(B, M, D) * q.reshape(B, 1, D)).reshape(B * M, D)
    di = jax.lax.broadcasted_iota(jnp.int32, (D, H), 0) // HD
    hi = jax.lax.broadcasted_iota(jnp.int32, (D, H), 1)
    hsel = jnp.where(di == hi, SCALE, 0.0).astype(jnp.float32)
    s3 = jnp.dot(P, hsel, preferred_element_type=jnp.float32).reshape(B, M, H)
    mx = jnp.max(s3, axis=1, keepdims=True)
    e = jnp.exp(s3 - mx)
    attn = e / jnp.sum(e, axis=1, keepdims=True)

    hi2 = jax.lax.broadcasted_iota(jnp.int32, (H, D), 0)
    di2 = jax.lax.broadcasted_iota(jnp.int32, (H, D), 1) // HD
    expand = jnp.where(hi2 == di2, 1.0, 0.0).astype(jnp.float32)
    A128 = jnp.dot(attn.reshape(B * M, H), expand,
                   preferred_element_type=jnp.float32)
    ctx = jnp.sum((A128 * V2).reshape(B, M, D), axis=1)

    tgt = jnp.dot(ctx, Wo[...], preferred_element_type=jnp.float32) + bo[...]
    a1 = jnp.maximum(
        jnp.dot(tgt, w1[...], preferred_element_type=jnp.float32) + b1[...], 0.0)
    hh = tgt + jnp.dot(a1, w2[...], preferred_element_type=jnp.float32) + b2[...]
    mu = jnp.mean(hh, axis=1, keepdims=True)
    dv = hh - mu
    var = jnp.mean(dv * dv, axis=1, keepdims=True)
    y = dv * jax.lax.rsqrt(var + 1e-5) * lg[...] + lb[...]

    t = jnp.dot(y, fw1[...], preferred_element_type=jnp.float32) + fb1[...]
    out_ref[...] = (
        jnp.dot(feat_ref[...], fw2[0:C, :], preferred_element_type=jnp.float32)
        + jnp.dot(t, fw2[C:2 * C, :], preferred_element_type=jnp.float32)
        + fb2[...])


def _bnorm_body(f_ref, g_ref, b_ref, out_ref):
    f = f_ref[...]
    mu = jnp.mean(f, axis=0, keepdims=True)
    dv = f - mu
    var = jnp.mean(dv * dv, axis=0, keepdims=True)
    out_ref[...] = jnp.maximum(
        dv * jax.lax.rsqrt(var + 1e-5) * g_ref[...] + b_ref[...], 0.0)


def _full(shape):
    return pl.BlockSpec(shape, lambda *_: tuple(0 for _ in shape))


def kernel(features, voxel_coords, pe_w1, pe_b1, pe_w2, pe_b2, proj_w, proj_b,
           Wq, bq, Wk, bk, Wv, bv, Wo, bo,
           ffn_w1, ffn_b1, ffn_w2, ffn_b2, ln_g, ln_b,
           fus_w1, fus_b1, fus_w2, fus_b2, bn_g, bn_b):
    r1 = lambda v: v.reshape(1, -1)
    coordsT = voxel_coords.T

    src, idx = pl.pallas_call(
        _prep_body,
        grid=(NB,),
        in_specs=[
            pl.BlockSpec((B, 3), lambda i: (i, 0)),
            _full((3, N)),
            pl.BlockSpec((B, C), lambda i: (i, 0)),
            _full(pe_w1.shape), _full((1, D // 2)),
            _full(pe_w2.shape), _full((1, D)),
            _full(proj_w.shape), _full((1, D)),
        ],
        out_specs=[
            pl.BlockSpec((B, D), lambda i: (i, 0)),
            pl.BlockSpec((B, M), lambda i: (i, 0)),
        ],
        out_shape=[
            jax.ShapeDtypeStruct((N, D), jnp.float32),
            jax.ShapeDtypeStruct((N, M), jnp.int32),
        ],
    )(voxel_coords, coordsT, features, pe_w1, r1(pe_b1), pe_w2, r1(pe_b2),
      proj_w, r1(proj_b))

    table = jnp.concatenate([src, jnp.zeros((TBL - N, D), jnp.float32)], axis=0)
    kvidx = idx.reshape(M, N).T
    gidx = jnp.concatenate([kvidx.reshape(N * M), idx[:, 0]])
    gout = _sc_gather(table, gidx)
    neigh, qrows = gout[:N * M], gout[N * M:]

    fused = pl.pallas_call(
        _main_body,
        grid=(NB,),
        in_specs=[
            pl.BlockSpec((B, M, D), lambda i: (i, 0, 0)),
            pl.BlockSpec((B, D), lambda i: (i, 0)),
            pl.BlockSpec((B, C), lambda i: (i, 0)),
            _full(Wq.shape), _full((1, D)),
            _full(Wk.shape), _full((1, D)),
            _full(Wv.shape), _full((1, D)),
            _full(Wo.shape), _full((1, D)),
            _full(ffn_w1.shape), _full((1, DFF)),
            _full(ffn_w2.shape), _full((1, D)),
            _full((1, D)), _full((1, D)),
            _full(fus_w1.shape), _full((1, C)),
            _full(fus_w2.shape), _full((1, C)),
        ],
        out_specs=pl.BlockSpec((B, C), lambda i: (i, 0)),
        out_shape=jax.ShapeDtypeStruct((N, C), jnp.float32),
    )(neigh.reshape(N, M, D), qrows, features,
      Wq, r1(bq), Wk, r1(bk), Wv, r1(bv), Wo, r1(bo),
      ffn_w1, r1(ffn_b1), ffn_w2, r1(ffn_b2), r1(ln_g), r1(ln_b),
      fus_w1, r1(fus_b1), fus_w2, r1(fus_b2))

    out = pl.pallas_call(
        _bnorm_body,
        in_specs=[_full((N, C)), _full((1, C)), _full((1, C))],
        out_specs=_full((N, C)),
        out_shape=jax.ShapeDtypeStruct((N, C), jnp.float32),
    )(fused, r1(bn_g), r1(bn_b))
    return out


# speedup vs baseline: 4.9777x; 1.0760x over previous
import jax
import jax.numpy as jnp
from jax.experimental import pallas as pl
from jax.experimental.pallas import tpu as pltpu
from jax.experimental.pallas import tpu_sc as plsc

N = 4096
C = 64
D = 128
M = 16
H = 4
HD = 32
DFF = 256
B = 256
NB = N // B
PAD = N
TBL = N + 8
BIG = 1 << 30
SCALE = 1.0 / (HD ** 0.5)


def _prep_body(coords_ref, coordsT_ref, feat_ref, pw1, pb1, pw2, pb2, prw, prb,
               src_ref, idx_ref):
    cb = coords_ref[...]
    crt = cb.astype(jnp.float32) * (1.0 / 39.0)
    h1 = jnp.maximum(
        jnp.dot(crt, pw1[...], preferred_element_type=jnp.float32) + pb1[...], 0.0)
    pe = jnp.dot(h1, pw2[...], preferred_element_type=jnp.float32) + pb2[...]
    src_ref[...] = (
        jnp.dot(feat_ref[...], prw[...], preferred_element_type=jnp.float32)
        + prb[...] + pe)

    cT = coordsT_ref[...]
    d = jnp.abs(cb[:, 0:1] - cT[0:1, :])
    d = d + jnp.abs(cb[:, 1:2] - cT[1:2, :])
    d = d + jnp.abs(cb[:, 2:3] - cT[2:3, :])
    j = jax.lax.broadcasted_iota(jnp.int32, (B, N), 1)
    key = jnp.where(d <= 4, d * N + j, BIG)
    for m in range(M):
        cur = jnp.min(key, axis=1, keepdims=True)
        idx_ref[:, m:m + 1] = jnp.where(cur < BIG,
                                        jnp.bitwise_and(cur, N - 1),
                                        PAD)
        key = jnp.where(key == cur, BIG, key)


NW = 32
CHUNK = 64
NBUF = 8
GTOT = N * M + N
ROWS_W = GTOT // NW
NCH = ROWS_W // CHUNK


N_PW = N * M // NW
Q_PW = N // NW
NCH_N = N_PW // CHUNK
NCH_Q = Q_PW // CHUNK


def _sc_gather(table, nidx, qidx):
    mesh = plsc.VectorSubcoreMesh(core_axis_name="c", subcore_axis_name="s")

    @pl.kernel(out_type=[jax.ShapeDtypeStruct((N * M, D), jnp.float32),
                         jax.ShapeDtypeStruct((N, D), jnp.float32)],
               mesh=mesh,
               scratch_types=[pltpu.VMEM((ROWS_W,), jnp.int32),
                              pltpu.VMEM((NBUF, CHUNK, D), jnp.float32),
                              pltpu.VMEM_SHARED((TBL, D), jnp.float32),
                              pltpu.SemaphoreType.DMA((NBUF,)),
                              pltpu.SemaphoreType.DMA((NBUF,))])
    def kern(x_hbm, ni_hbm, qi_hbm, no_hbm, qo_hbm, idx_v, rows, tbl_spm,
             gsem, wsem):
        sid = jax.lax.axis_index("s")
        wid = sid * 2 + jax.lax.axis_index("c")
        nb = wid * N_PW
        qb = wid * Q_PW

        @pl.when(sid == 0)
        def _():
            pltpu.sync_copy(x_hbm, tbl_spm)

        plsc.subcore_barrier()
        pltpu.sync_copy(ni_hbm.at[pl.ds(nb, N_PW)], idx_v.at[pl.ds(0, N_PW)])
        pltpu.sync_copy(qi_hbm.at[pl.ds(qb, Q_PW)],
                        idx_v.at[pl.ds(N_PW, Q_PW)])

        def dst(s):
            if s < NCH_N:
                return no_hbm.at[pl.ds(nb + s * CHUNK, CHUNK)]
            return qo_hbm.at[pl.ds(qb + (s - NCH_N) * CHUNK, CHUNK)]

        wb = [None] * NBUF
        for g in range(0, NCH, NBUF):
            fired = []
            for s in range(g, min(g + NBUF, NCH)):
                b = s % NBUF
                if wb[b] is not None:
                    wb[b].wait()
                h = pltpu.async_copy(
                    tbl_spm.at[idx_v.at[pl.ds(s * CHUNK, CHUNK)]],
                    rows.at[b], gsem.at[b])
                fired.append((s, b, h))
            for s, b, h in fired:
                h.wait()
                wb[b] = pltpu.async_copy(rows.at[b], dst(s), wsem.at[b])
        for b in range(NBUF):
            if wb[b] is not None:
                wb[b].wait()

    return kern(table, nidx, qidx)


def _main_body(neigh_ref, q_ref, feat_ref, Wq, bq, Wk, bk, Wv, bv, Wo, bo,
               w1, b1, w2, b2, lg, lb, fw1, fb1, fw2, fb2, out_ref):
    n2 = neigh_ref[...].reshape(B * M, D)
    K2 = jnp.dot(n2, Wk[...], preferred_element_type=jnp.float32) + bk[...]
    V2 = jnp.dot(n2, Wv[...], preferred_element_type=jnp.float32) + bv[...]
    q = jnp.dot(q_ref[...], Wq[...], preferred_element_type=jnp.float32) + bq[...]

    P = (K2.reshape(B, M, D) * q.reshape(B, 1, D)).reshape(B * M, D)
    di = jax.lax.broadcasted_iota(jnp.int32, (D, H), 0) // HD
    hi = jax.lax.broadcasted_iota(jnp.int32, (D, H), 1)
    hsel = jnp.where(di == hi, SCALE, 0.0).astype(jnp.float32)
    s3 = jnp.dot(P, hsel, preferred_element_type=jnp.float32).reshape(B, M, H)
    mx = jnp.max(s3, axis=1, keepdims=True)
    e = jnp.exp(s3 - mx)
    attn = e / jnp.sum(e, axis=1, keepdims=True)

    hi2 = jax.lax.broadcasted_iota(jnp.int32, (H, D), 0)
    di2 = jax.lax.broadcasted_iota(jnp.int32, (H, D), 1) // HD
    expand = jnp.where(hi2 == di2, 1.0, 0.0).astype(jnp.float32)
    A128 = jnp.dot(attn.reshape(B * M, H), expand,
                   preferred_element_type=jnp.float32)
    ctx = jnp.sum((A128 * V2).reshape(B, M, D), axis=1)

    tgt = jnp.dot(ctx, Wo[...], preferred_element_type=jnp.float32) + bo[...]
    a1 = jnp.maximum(
        jnp.dot(tgt, w1[...], preferred_element_type=jnp.float32) + b1[...], 0.0)
    hh = tgt + jnp.dot(a1, w2[...], preferred_element_type=jnp.float32) + b2[...]
    mu = jnp.mean(hh, axis=1, keepdims=True)
    dv = hh - mu
    var = jnp.mean(dv * dv, axis=1, keepdims=True)
    y = dv * jax.lax.rsqrt(var + 1e-5) * lg[...] + lb[...]

    t = jnp.dot(y, fw1[...], preferred_element_type=jnp.float32) + fb1[...]
    out_ref[...] = (
        jnp.dot(feat_ref[...], fw2[0:C, :], preferred_element_type=jnp.float32)
        + jnp.dot(t, fw2[C:2 * C, :], preferred_element_type=jnp.float32)
        + fb2[...])


def _bnorm_body(f_ref, g_ref, b_ref, out_ref):
    f = f_ref[...]
    mu = jnp.mean(f, axis=0, keepdims=True)
    dv = f - mu
    var = jnp.mean(dv * dv, axis=0, keepdims=True)
    out_ref[...] = jnp.maximum(
        dv * jax.lax.rsqrt(var + 1e-5) * g_ref[...] + b_ref[...], 0.0)


def _full(shape):
    return pl.BlockSpec(shape, lambda *_: tuple(0 for _ in shape))


def kernel(features, voxel_coords, pe_w1, pe_b1, pe_w2, pe_b2, proj_w, proj_b,
           Wq, bq, Wk, bk, Wv, bv, Wo, bo,
           ffn_w1, ffn_b1, ffn_w2, ffn_b2, ln_g, ln_b,
           fus_w1, fus_b1, fus_w2, fus_b2, bn_g, bn_b):
    r1 = lambda v: v.reshape(1, -1)
    coordsT = voxel_coords.T

    src, idx = pl.pallas_call(
        _prep_body,
        grid=(NB,),
        in_specs=[
            pl.BlockSpec((B, 3), lambda i: (i, 0)),
            _full((3, N)),
            pl.BlockSpec((B, C), lambda i: (i, 0)),
            _full(pe_w1.shape), _full((1, D // 2)),
            _full(pe_w2.shape), _full((1, D)),
            _full(proj_w.shape), _full((1, D)),
        ],
        out_specs=[
            pl.BlockSpec((B, D), lambda i: (i, 0)),
            pl.BlockSpec((B, M), lambda i: (i, 0)),
        ],
        out_shape=[
            jax.ShapeDtypeStruct((N, D), jnp.float32),
            jax.ShapeDtypeStruct((N, M), jnp.int32),
        ],
    )(voxel_coords, coordsT, features, pe_w1, r1(pe_b1), pe_w2, r1(pe_b2),
      proj_w, r1(proj_b))

    table = jnp.concatenate([src, jnp.zeros((TBL - N, D), jnp.float32)], axis=0)
    kvidx = idx.reshape(M, N).T
    neigh, qrows = _sc_gather(table, kvidx.reshape(N * M), idx[:, 0])

    fused = pl.pallas_call(
        _main_body,
        grid=(NB,),
        in_specs=[
            pl.BlockSpec((B, M, D), lambda i: (i, 0, 0)),
            pl.BlockSpec((B, D), lambda i: (i, 0)),
            pl.BlockSpec((B, C), lambda i: (i, 0)),
            _full(Wq.shape), _full((1, D)),
            _full(Wk.shape), _full((1, D)),
            _full(Wv.shape), _full((1, D)),
            _full(Wo.shape), _full((1, D)),
            _full(ffn_w1.shape), _full((1, DFF)),
            _full(ffn_w2.shape), _full((1, D)),
            _full((1, D)), _full((1, D)),
            _full(fus_w1.shape), _full((1, C)),
            _full(fus_w2.shape), _full((1, C)),
        ],
        out_specs=pl.BlockSpec((B, C), lambda i: (i, 0)),
        out_shape=jax.ShapeDtypeStruct((N, C), jnp.float32),
    )(neigh.reshape(N, M, D), qrows, features,
      Wq, r1(bq), Wk, r1(bk), Wv, r1(bv), Wo, r1(bo),
      ffn_w1, r1(ffn_b1), ffn_w2, r1(ffn_b2), r1(ln_g), r1(ln_b),
      fus_w1, r1(fus_b1), fus_w2, r1(fus_b2))

    out = pl.pallas_call(
        _bnorm_body,
        in_specs=[_full((N, C)), _full((1, C)), _full((1, C))],
        out_specs=_full((N, C)),
        out_shape=jax.ShapeDtypeStruct((N, C), jnp.float32),
    )(fused, r1(bn_g), r1(bn_b))
    return out


# speedup vs baseline: 5.6771x; 1.1405x over previous
import jax
import jax.numpy as jnp
from jax.experimental import pallas as pl
from jax.experimental.pallas import tpu as pltpu
from jax.experimental.pallas import tpu_sc as plsc

N = 4096
C = 64
D = 128
M = 16
H = 4
HD = 32
DFF = 256
B = 256
NB = N // B
PAD = N
TBL = N + 8
BIGF = 1e9
SCALE = 1.0 / (HD ** 0.5)


def _prep_body(coords_ref, coordsT_ref, feat_ref, pw1, pb1, pw2, pb2, prw, prb,
               src_ref, idx_ref):
    cb = coords_ref[...]
    crt = cb.astype(jnp.float32) * (1.0 / 39.0)
    h1 = jnp.maximum(
        jnp.dot(crt, pw1[...], preferred_element_type=jnp.float32) + pb1[...], 0.0)
    pe = jnp.dot(h1, pw2[...], preferred_element_type=jnp.float32) + pb2[...]
    src_ref[...] = (
        jnp.dot(feat_ref[...], prw[...], preferred_element_type=jnp.float32)
        + prb[...] + pe)

    cTf = coordsT_ref[...].astype(jnp.float32)
    jf = jax.lax.broadcasted_iota(jnp.int32, (8, N), 1).astype(jnp.float32)
    cbf = cb.astype(jnp.float32)
    for g in range(B // 8):
        cbg = cbf[g * 8:(g + 1) * 8, :]
        dg = jnp.abs(cbg[:, 0:1] - cTf[0:1, :])
        dg = dg + jnp.abs(cbg[:, 1:2] - cTf[1:2, :])
        dg = dg + jnp.abs(cbg[:, 2:3] - cTf[2:3, :])
        keyg = jnp.where(dg <= 4.0, dg * float(N) + jf, BIGF)
        for m in range(M):
            cur = jnp.min(keyg, axis=1, keepdims=True)
            curi = cur.astype(jnp.int32)
            idx_ref[g * 8:(g + 1) * 8, m:m + 1] = jnp.where(
                cur < BIGF, jnp.bitwise_and(curi, N - 1), PAD)
            keyg = jnp.where(keyg == cur, BIGF, keyg)


NW = 32
CHUNK = 64
NBUF = 8
GTOT = N * M + N
ROWS_W = GTOT // NW
NCH = ROWS_W // CHUNK


N_PW = N * M // NW
Q_PW = N // NW
NCH_N = N_PW // CHUNK
NCH_Q = Q_PW // CHUNK


def _sc_gather(table, nidx, qidx):
    mesh = plsc.VectorSubcoreMesh(core_axis_name="c", subcore_axis_name="s")

    @pl.kernel(out_type=[jax.ShapeDtypeStruct((N * M, D), jnp.float32),
                         jax.ShapeDtypeStruct((N, D), jnp.float32)],
               mesh=mesh,
               scratch_types=[pltpu.VMEM((ROWS_W,), jnp.int32),
                              pltpu.VMEM((NBUF, CHUNK, D), jnp.float32),
                              pltpu.VMEM_SHARED((TBL, D), jnp.float32),
                              pltpu.SemaphoreType.DMA((NBUF,)),
                              pltpu.SemaphoreType.DMA((NBUF,))])
    def kern(x_hbm, ni_hbm, qi_hbm, no_hbm, qo_hbm, idx_v, rows, tbl_spm,
             gsem, wsem):
        sid = jax.lax.axis_index("s")
        wid = sid * 2 + jax.lax.axis_index("c")
        nb = wid * N_PW
        qb = wid * Q_PW

        @pl.when(sid == 0)
        def _():
            pltpu.sync_copy(x_hbm, tbl_spm)

        plsc.subcore_barrier()
        pltpu.sync_copy(ni_hbm.at[pl.ds(nb, N_PW)], idx_v.at[pl.ds(0, N_PW)])
        pltpu.sync_copy(qi_hbm.at[pl.ds(qb, Q_PW)],
                        idx_v.at[pl.ds(N_PW, Q_PW)])

        def dst(s):
            if s < NCH_N:
                return no_hbm.at[pl.ds(nb + s * CHUNK, CHUNK)]
            return qo_hbm.at[pl.ds(qb + (s - NCH_N) * CHUNK, CHUNK)]

        wb = [None] * NBUF
        for g in range(0, NCH, NBUF):
            fired = []
            for s in range(g, min(g + NBUF, NCH)):
                b = s % NBUF
                if wb[b] is not None:
                    wb[b].wait()
                h = pltpu.async_copy(
                    tbl_spm.at[idx_v.at[pl.ds(s * CHUNK, CHUNK)]],
                    rows.at[b], gsem.at[b])
                fired.append((s, b, h))
            for s, b, h in fired:
                h.wait()
                wb[b] = pltpu.async_copy(rows.at[b], dst(s), wsem.at[b])
        for b in range(NBUF):
            if wb[b] is not None:
                wb[b].wait()

    return kern(table, nidx, qidx)


def _main_body(neigh_ref, q_ref, feat_ref, Wq, bq, Wk, bk, Wv, bv, Wo, bo,
               w1, b1, w2, b2, lg, lb, fw1, fb1, fw2, fb2, out_ref):
    n2 = neigh_ref[...].reshape(B * M, D)
    K2 = jnp.dot(n2, Wk[...], preferred_element_type=jnp.float32) + bk[...]
    V2 = jnp.dot(n2, Wv[...], preferred_element_type=jnp.float32) + bv[...]
    q = jnp.dot(q_ref[...], Wq[...], preferred_element_type=jnp.float32) + bq[...]

    P = (K2.reshape(B, M, D) * q.reshape(B, 1, D)).reshape(B * M, D)
    di = jax.lax.broadcasted_iota(jnp.int32, (D, H), 0) // HD
    hi = jax.lax.broadcasted_iota(jnp.int32, (D, H), 1)
    hsel = jnp.where(di == hi, SCALE, 0.0).astype(jnp.float32)
    s3 = jnp.dot(P, hsel, preferred_element_type=jnp.float32).reshape(B, M, H)
    mx = jnp.max(s3, axis=1, keepdims=True)
    e = jnp.exp(s3 - mx)
    attn = e / jnp.sum(e, axis=1, keepdims=True)

    hi2 = jax.lax.broadcasted_iota(jnp.int32, (H, D), 0)
    di2 = jax.lax.broadcasted_iota(jnp.int32, (H, D), 1) // HD
    expand = jnp.where(hi2 == di2, 1.0, 0.0).astype(jnp.float32)
    A128 = jnp.dot(attn.reshape(B * M, H), expand,
                   preferred_element_type=jnp.float32)
    ctx = jnp.sum((A128 * V2).reshape(B, M, D), axis=1)

    tgt = jnp.dot(ctx, Wo[...], preferred_element_type=jnp.float32) + bo[...]
    a1 = jnp.maximum(
        jnp.dot(tgt, w1[...], preferred_element_type=jnp.float32) + b1[...], 0.0)
    hh = tgt + jnp.dot(a1, w2[...], preferred_element_type=jnp.float32) + b2[...]
    mu = jnp.mean(hh, axis=1, keepdims=True)
    dv = hh - mu
    var = jnp.mean(dv * dv, axis=1, keepdims=True)
    y = dv * jax.lax.rsqrt(var + 1e-5) * lg[...] + lb[...]

    t = jnp.dot(y, fw1[...], preferred_element_type=jnp.float32) + fb1[...]
    out_ref[...] = (
        jnp.dot(feat_ref[...], fw2[0:C, :], preferred_element_type=jnp.float32)
        + jnp.dot(t, fw2[C:2 * C, :], preferred_element_type=jnp.float32)
        + fb2[...])


def _bnorm_body(f_ref, g_ref, b_ref, out_ref):
    f = f_ref[...]
    mu = jnp.mean(f, axis=0, keepdims=True)
    dv = f - mu
    var = jnp.mean(dv * dv, axis=0, keepdims=True)
    out_ref[...] = jnp.maximum(
        dv * jax.lax.rsqrt(var + 1e-5) * g_ref[...] + b_ref[...], 0.0)


def _full(shape):
    return pl.BlockSpec(shape, lambda *_: tuple(0 for _ in shape))


def kernel(features, voxel_coords, pe_w1, pe_b1, pe_w2, pe_b2, proj_w, proj_b,
           Wq, bq, Wk, bk, Wv, bv, Wo, bo,
           ffn_w1, ffn_b1, ffn_w2, ffn_b2, ln_g, ln_b,
           fus_w1, fus_b1, fus_w2, fus_b2, bn_g, bn_b):
    r1 = lambda v: v.reshape(1, -1)
    coordsT = voxel_coords.T

    src, idx = pl.pallas_call(
        _prep_body,
        grid=(NB,),
        in_specs=[
            pl.BlockSpec((B, 3), lambda i: (i, 0)),
            _full((3, N)),
            pl.BlockSpec((B, C), lambda i: (i, 0)),
            _full(pe_w1.shape), _full((1, D // 2)),
            _full(pe_w2.shape), _full((1, D)),
            _full(proj_w.shape), _full((1, D)),
        ],
        out_specs=[
            pl.BlockSpec((B, D), lambda i: (i, 0)),
            pl.BlockSpec((B, M), lambda i: (i, 0)),
        ],
        out_shape=[
            jax.ShapeDtypeStruct((N, D), jnp.float32),
            jax.ShapeDtypeStruct((N, M), jnp.int32),
        ],
    )(voxel_coords, coordsT, features, pe_w1, r1(pe_b1), pe_w2, r1(pe_b2),
      proj_w, r1(proj_b))

    table = jnp.concatenate([src, jnp.zeros((TBL - N, D), jnp.float32)], axis=0)
    kvidx = idx.reshape(M, N).T
    neigh, qrows = _sc_gather(table, kvidx.reshape(N * M), idx[:, 0])

    fused = pl.pallas_call(
        _main_body,
        grid=(NB,),
        in_specs=[
            pl.BlockSpec((B, M, D), lambda i: (i, 0, 0)),
            pl.BlockSpec((B, D), lambda i: (i, 0)),
            pl.BlockSpec((B, C), lambda i: (i, 0)),
            _full(Wq.shape), _full((1, D)),
            _full(Wk.shape), _full((1, D)),
            _full(Wv.shape), _full((1, D)),
            _full(Wo.shape), _full((1, D)),
            _full(ffn_w1.shape), _full((1, DFF)),
            _full(ffn_w2.shape), _full((1, D)),
            _full((1, D)), _full((1, D)),
            _full(fus_w1.shape), _full((1, C)),
            _full(fus_w2.shape), _full((1, C)),
        ],
        out_specs=pl.BlockSpec((B, C), lambda i: (i, 0)),
        out_shape=jax.ShapeDtypeStruct((N, C), jnp.float32),
    )(neigh.reshape(N, M, D), qrows, features,
      Wq, r1(bq), Wk, r1(bk), Wv, r1(bv), Wo, r1(bo),
      ffn_w1, r1(ffn_b1), ffn_w2, r1(ffn_b2), r1(ln_g), r1(ln_b),
      fus_w1, r1(fus_b1), fus_w2, r1(fus_b2))

    out = pl.pallas_call(
        _bnorm_body,
        in_specs=[_full((N, C)), _full((1, C)), _full((1, C))],
        out_specs=_full((N, C)),
        out_shape=jax.ShapeDtypeStruct((N, C), jnp.float32),
    )(fused, r1(bn_g), r1(bn_b))
    return out
